# Initial kernel scaffold; baseline (speedup 1.0000x reference)
import jax
import jax.numpy as jnp
from jax.experimental import pallas as pl


def kernel(N, CA, CB, coeff, cutoffs, mask):
    raise NotImplementedError("write your pallas kernel here")


# speedup vs baseline: 51.6868x; 51.6868x over previous
import math

import jax
import jax.numpy as jnp
from jax.experimental import pallas as pl
from jax.experimental.pallas import tpu as pltpu

_L = 512
_NK = 25
_ROWS = 8
_TWO_PI = 2.0 * math.pi


def _cross(a, b):
    ax, ay, az = a
    bx, by, bz = b
    return (ay * bz - az * by, az * bx - ax * bz, ax * by - ay * bx)


def _body(cut_ref, ni_ref, cai_ref, cbi_ref, cbj_ref, coeff_ref, mask_ref,
          out_ref):
    step_idx = pl.program_id(0)
    c0 = cut_ref[0, 0]
    h = cut_ref[0, 1] - cut_ref[0, 0]
    rh = 1.0 / h
    h2_6 = h * h * (1.0 / 6.0)

    n = ni_ref[...]
    ca = cai_ref[...]
    cb = cbi_ref[...]
    nc = (n[0], n[1], n[2])
    cac = (ca[0], ca[1], ca[2])
    cbc = (cb[0], cb[1], cb[2])
    b0 = tuple(cac[k] - nc[k] for k in range(3))
    b1 = tuple(cbc[k] - cac[k] for k in range(3))
    n1 = _cross(b0, b1)
    A = _cross(n1, b1)
    nrm = jnp.sqrt(b1[0] * b1[0] + b1[1] * b1[1] + b1[2] * b1[2]) + 1e-9
    Braw = _cross(A, b1)
    Bv = tuple(Braw[k] / nrm for k in range(3))
    cx = -(A[0] * cbc[0] + A[1] * cbc[1] + A[2] * cbc[2])
    cy = -(Bv[0] * cbc[0] + Bv[1] * cbc[1] + Bv[2] * cbc[2])

    cbj = cbj_ref[...]
    cm = coeff_ref[...]
    mf = mask_ref[...]

    nb = ni_ref.shape[2]
    vsum = jnp.zeros(mf.shape, jnp.float32)
    for b in range(nb):
        ax = A[0][:, b:b + 1]
        ay = A[1][:, b:b + 1]
        az = A[2][:, b:b + 1]
        bx = Bv[0][:, b:b + 1]
        by = Bv[1][:, b:b + 1]
        bz = Bv[2][:, b:b + 1]
        jx = cbj[0, b][None, :]
        jy = cbj[1, b][None, :]
        jz = cbj[2, b][None, :]
        X = ax * jx + ay * jy + az * jz + cx[:, b:b + 1]
        Y = bx * jx + by * jy + bz * jz + cy[:, b:b + 1]
        theta = jnp.arctan2(Y, X)
        tw = jnp.where(theta < c0, theta + _TWO_PI, theta)
        q = (tw - c0) * rh
        bif = jnp.clip(jnp.floor(q), 0.0, float(_NK - 2))
        bi = bif.astype(jnp.int32)
        u = q - bif
        t = 1.0 - u
        idx = bi[..., None]
        y_lo = jnp.take_along_axis(cm, idx, axis=2)[..., 0]
        y_hi = jnp.take_along_axis(cm, idx + 1, axis=2)[..., 0]
        m_lo = jnp.take_along_axis(cm, idx + _NK, axis=2)[..., 0]
        m_hi = jnp.take_along_axis(cm, idx + (_NK + 1), axis=2)[..., 0]
        val = (t * y_lo + u * y_hi +
               ((t * t * t - t) * m_lo + (u * u * u - u) * m_hi) * h2_6)
        vsum = vsum + val * mf

    partial = jnp.sum(vsum)[None, None]

    @pl.when(step_idx == 0)
    def _():
        out_ref[...] = jnp.zeros((1, 1), jnp.float32)

    out_ref[...] += partial


def kernel(N, CA, CB, coeff, cutoffs, mask):
    L = mask.shape[0]
    nb = N.shape[0]
    ni = jnp.transpose(N, (2, 1, 0))
    cai = jnp.transpose(CA, (2, 1, 0))
    cbi = jnp.transpose(CB, (2, 1, 0))
    cbj = jnp.transpose(CB, (2, 0, 1))
    c2 = coeff.reshape(L, L, 2 * _NK)
    mf = mask.astype(jnp.float32)
    cuts = cutoffs.reshape(1, _NK)

    out = pl.pallas_call(
        _body,
        grid=(L // _ROWS,),
        in_specs=[
            pl.BlockSpec(memory_space=pltpu.SMEM),
            pl.BlockSpec((3, _ROWS, nb), lambda i: (0, i, 0)),
            pl.BlockSpec((3, _ROWS, nb), lambda i: (0, i, 0)),
            pl.BlockSpec((3, _ROWS, nb), lambda i: (0, i, 0)),
            pl.BlockSpec((3, nb, L), lambda i: (0, 0, 0)),
            pl.BlockSpec((_ROWS, L, 2 * _NK), lambda i: (i, 0, 0)),
            pl.BlockSpec((_ROWS, L), lambda i: (i, 0)),
        ],
        out_specs=pl.BlockSpec((1, 1), lambda i: (0, 0)),
        out_shape=jax.ShapeDtypeStruct((1, 1), jnp.float32),
        compiler_params=pltpu.CompilerParams(
            dimension_semantics=("arbitrary",)),
    )(cuts, ni, cai, cbi, cbj, c2, mf)
    return out[0, 0]


# speedup vs baseline: 221.8939x; 4.2931x over previous
import math

import jax
import jax.numpy as jnp
from jax.experimental import pallas as pl
from jax.experimental.pallas import tpu as pltpu

_L = 512
_NK = 25
_ROWS = 8
_TWO_PI = 2.0 * math.pi


def _cross(a, b):
    ax, ay, az = a
    bx, by, bz = b
    return (ay * bz - az * by, az * bx - ax * bz, ax * by - ay * bx)


def _body(cut_ref, ni_ref, cai_ref, cbi_ref, cbj_ref, coeff_ref, mask_ref,
          out_ref):
    step_idx = pl.program_id(0)
    c0 = cut_ref[0, 0]
    h = cut_ref[0, 1] - cut_ref[0, 0]
    rh = 1.0 / h
    h2_6 = h * h * (1.0 / 6.0)

    n = ni_ref[...]
    ca = cai_ref[...]
    cb = cbi_ref[...]
    nc = (n[0], n[1], n[2])
    cac = (ca[0], ca[1], ca[2])
    cbc = (cb[0], cb[1], cb[2])
    b0 = tuple(cac[k] - nc[k] for k in range(3))
    b1 = tuple(cbc[k] - cac[k] for k in range(3))
    n1 = _cross(b0, b1)
    A = _cross(n1, b1)
    nrm = jnp.sqrt(b1[0] * b1[0] + b1[1] * b1[1] + b1[2] * b1[2]) + 1e-9
    Braw = _cross(A, b1)
    Bv = tuple(Braw[k] / nrm for k in range(3))
    cx = -(A[0] * cbc[0] + A[1] * cbc[1] + A[2] * cbc[2])
    cy = -(Bv[0] * cbc[0] + Bv[1] * cbc[1] + Bv[2] * cbc[2])

    cbj = cbj_ref[...]
    cm = coeff_ref[...]
    mf = mask_ref[...]

    nb = ni_ref.shape[2]
    rows = mf.shape[0]
    ncols = mf.shape[1]
    kf = jax.lax.broadcasted_iota(
        jnp.int32, (rows, ncols, 2 * _NK), 2).astype(jnp.float32)
    km = kf - float(_NK)
    wsum = jnp.zeros((rows, ncols, 2 * _NK), jnp.float32)
    for b in range(nb):
        ax = A[0][:, b:b + 1]
        ay = A[1][:, b:b + 1]
        az = A[2][:, b:b + 1]
        bx = Bv[0][:, b:b + 1]
        by = Bv[1][:, b:b + 1]
        bz = Bv[2][:, b:b + 1]
        jx = cbj[0, b][None, :]
        jy = cbj[1, b][None, :]
        jz = cbj[2, b][None, :]
        X = ax * jx + ay * jy + az * jz + cx[:, b:b + 1]
        Y = bx * jx + by * jy + bz * jz + cy[:, b:b + 1]
        theta = jnp.arctan2(Y, X)
        q2 = (jnp.where(theta < c0, theta + _TWO_PI, theta) - c0) * rh
        u = q2 - jnp.floor(q2)
        stu = (u * u - u) * h2_6
        q = q2[:, :, None]
        d1 = jnp.abs(kf - q)
        hat = jnp.maximum(1.0 - d1, 0.0)
        d2 = jnp.abs(km - q)
        wm = jnp.where(d2 < 1.0, (2.0 - d2) * stu[:, :, None], 0.0)
        wsum = wsum + (hat + wm)

    pair_vals = jnp.sum(wsum * cm, axis=2)
    partial = jnp.sum(pair_vals * mf)[None, None]

    @pl.when(step_idx == 0)
    def _():
        out_ref[...] = jnp.zeros((1, 1), jnp.float32)

    out_ref[...] += partial


def kernel(N, CA, CB, coeff, cutoffs, mask):
    L = mask.shape[0]
    nb = N.shape[0]
    ni = jnp.transpose(N, (2, 1, 0))
    cai = jnp.transpose(CA, (2, 1, 0))
    cbi = jnp.transpose(CB, (2, 1, 0))
    cbj = jnp.transpose(CB, (2, 0, 1))
    c2 = coeff.reshape(L, L, 2 * _NK)
    mf = mask.astype(jnp.float32)
    cuts = cutoffs.reshape(1, _NK)

    out = pl.pallas_call(
        _body,
        grid=(L // _ROWS,),
        in_specs=[
            pl.BlockSpec(memory_space=pltpu.SMEM),
            pl.BlockSpec((3, _ROWS, nb), lambda i: (0, i, 0)),
            pl.BlockSpec((3, _ROWS, nb), lambda i: (0, i, 0)),
            pl.BlockSpec((3, _ROWS, nb), lambda i: (0, i, 0)),
            pl.BlockSpec((3, nb, L), lambda i: (0, 0, 0)),
            pl.BlockSpec((_ROWS, L, 2 * _NK), lambda i: (i, 0, 0)),
            pl.BlockSpec((_ROWS, L), lambda i: (i, 0)),
        ],
        out_specs=pl.BlockSpec((1, 1), lambda i: (0, 0)),
        out_shape=jax.ShapeDtypeStruct((1, 1), jnp.float32),
        compiler_params=pltpu.CompilerParams(
            dimension_semantics=("arbitrary",)),
    )(cuts, ni, cai, cbi, cbj, c2, mf)
    return out[0, 0]


# speedup vs baseline: 993.7187x; 4.4784x over previous
import math

import jax
import jax.numpy as jnp
from jax.experimental import pallas as pl
from jax.experimental.pallas import tpu as pltpu

_L = 512
_NK = 25
_ROWS = 8
_TWO_PI = 2.0 * math.pi


def _cross(a, b):
    ax, ay, az = a
    bx, by, bz = b
    return (ay * bz - az * by, az * bx - ax * bz, ax * by - ay * bx)


def _body(cut_ref, ni_ref, cai_ref, cbi_ref, cbj_ref, coeff_ref, mask_ref,
          out_ref):
    step_idx = pl.program_id(0)
    c0 = cut_ref[0, 0]
    h = cut_ref[0, 1] - cut_ref[0, 0]
    rh = 1.0 / h
    h2_6 = h * h * (1.0 / 6.0)

    n = ni_ref[...]
    ca = cai_ref[...]
    cb = cbi_ref[...]
    nc = (n[0], n[1], n[2])
    cac = (ca[0], ca[1], ca[2])
    cbc = (cb[0], cb[1], cb[2])
    b0 = tuple(cac[k] - nc[k] for k in range(3))
    b1 = tuple(cbc[k] - cac[k] for k in range(3))
    n1 = _cross(b0, b1)
    A = _cross(n1, b1)
    nrm = jnp.sqrt(b1[0] * b1[0] + b1[1] * b1[1] + b1[2] * b1[2]) + 1e-9
    Braw = _cross(A, b1)
    Bv = tuple(Braw[k] / nrm for k in range(3))
    cx = -(A[0] * cbc[0] + A[1] * cbc[1] + A[2] * cbc[2])
    cy = -(Bv[0] * cbc[0] + Bv[1] * cbc[1] + Bv[2] * cbc[2])

    cbj = cbj_ref[...]
    mf = mask_ref[...]

    nb = ni_ref.shape[2]
    acc = jnp.zeros(mf.shape, jnp.float32)
    for b in range(nb):
        ax = A[0][:, b:b + 1]
        ay = A[1][:, b:b + 1]
        az = A[2][:, b:b + 1]
        bx = Bv[0][:, b:b + 1]
        by = Bv[1][:, b:b + 1]
        bz = Bv[2][:, b:b + 1]
        jx = cbj[0, b][None, :]
        jy = cbj[1, b][None, :]
        jz = cbj[2, b][None, :]
        X = ax * jx + ay * jy + az * jz + cx[:, b:b + 1]
        Y = bx * jx + by * jy + bz * jz + cy[:, b:b + 1]
        theta = jnp.arctan2(Y, X)
        q = (jnp.where(theta < c0, theta + _TWO_PI, theta) - c0) * rh
        u = q - jnp.floor(q)
        stu = (u * u - u) * h2_6
        for k in range(_NK):
            g = 1.0 - jnp.abs(q - float(k))
            p = jnp.maximum(g, 0.0)
            w2 = (p + jnp.sign(p)) * stu
            acc = acc + p * coeff_ref[k] + w2 * coeff_ref[k + _NK]

    partial = jnp.sum(acc * mf)[None, None]

    @pl.when(step_idx == 0)
    def _():
        out_ref[...] = jnp.zeros((1, 1), jnp.float32)

    out_ref[...] += partial


def kernel(N, CA, CB, coeff, cutoffs, mask):
    L = mask.shape[0]
    nb = N.shape[0]
    ni = jnp.transpose(N, (2, 1, 0))
    cai = jnp.transpose(CA, (2, 1, 0))
    cbi = jnp.transpose(CB, (2, 1, 0))
    cbj = jnp.transpose(CB, (2, 0, 1))
    c2 = jnp.transpose(coeff.reshape(L, L, 2 * _NK), (2, 0, 1))
    mf = mask.astype(jnp.float32)
    cuts = cutoffs.reshape(1, _NK)

    out = pl.pallas_call(
        _body,
        grid=(L // _ROWS,),
        in_specs=[
            pl.BlockSpec(memory_space=pltpu.SMEM),
            pl.BlockSpec((3, _ROWS, nb), lambda i: (0, i, 0)),
            pl.BlockSpec((3, _ROWS, nb), lambda i: (0, i, 0)),
            pl.BlockSpec((3, _ROWS, nb), lambda i: (0, i, 0)),
            pl.BlockSpec((3, nb, L), lambda i: (0, 0, 0)),
            pl.BlockSpec((2 * _NK, _ROWS, L), lambda i: (0, i, 0)),
            pl.BlockSpec((_ROWS, L), lambda i: (i, 0)),
        ],
        out_specs=pl.BlockSpec((1, 1), lambda i: (0, 0)),
        out_shape=jax.ShapeDtypeStruct((1, 1), jnp.float32),
        compiler_params=pltpu.CompilerParams(
            dimension_semantics=("arbitrary",)),
    )(cuts, ni, cai, cbi, cbj, c2, mf)
    return out[0, 0]
